# Initial kernel scaffold; baseline (speedup 1.0000x reference)
import jax
import jax.numpy as jnp
from jax.experimental import pallas as pl


def kernel(x, edge_index, W1, b1, W2, b2, Wfc, bfc):
    raise NotImplementedError("write your pallas kernel here")


# speedup vs baseline: 21.7240x; 21.7240x over previous
import functools

import jax
import jax.numpy as jnp
from jax import lax
from jax.experimental import pallas as pl
from jax.experimental.pallas import tpu as pltpu
from jax.experimental.pallas import tpu_sc as plsc

N = 10000
E = 320000
IN_DIM = 128
HID = 16
OUT = 8

NC = 2
NS = 16
NW = NC * NS
CH = 128
EPW = 10240
EP = NW * EPW
NCH = EPW // CH
NP = 10240
RPS = NP // NS

_mesh = plsc.VectorSubcoreMesh(
    core_axis_name="c", subcore_axis_name="s", num_cores=NC, num_subcores=NS
)
_sc_params = pltpu.CompilerParams(use_tc_tiling_on_sc=False)


def _sc_degree_body(dst_hbm, zeros_hbm, out_hbm, idx_v, ones_v, slab_v, deg_sh):
    c = lax.axis_index("c")
    s = lax.axis_index("s")
    wid = s * NC + c
    rs = pl.ds(s * RPS, RPS)
    pltpu.sync_copy(zeros_hbm.at[rs], slab_v)
    pltpu.sync_copy(slab_v, deg_sh.at[rs])
    for j in range(CH // 16):
        ones_v[pl.ds(j * 16, 16)] = jnp.ones((16,), jnp.float32)
    plsc.subcore_barrier()

    def loop(g, carry):
        base = wid * EPW + g * CH
        pltpu.sync_copy(dst_hbm.at[pl.ds(base, CH)], idx_v)
        pltpu.sync_copy(ones_v, deg_sh.at[idx_v], add=True)
        return carry

    lax.fori_loop(0, NCH, loop, 0)
    plsc.subcore_barrier()
    pltpu.sync_copy(deg_sh.at[rs], slab_v)
    pltpu.sync_copy(slab_v, out_hbm.at[pl.ds(c * NP + s * RPS, RPS)])


def _make_sc_scatter_body(D):
    def body(src_hbm, dst_hbm, h_hbm, zeros_hbm, out_hbm,
             isrc_v, idst_v, rows_v, acc_sh, sem):
        c = lax.axis_index("c")
        s = lax.axis_index("s")
        wid = s * NC + c

        @pl.when(s == 0)
        def _init():
            pltpu.sync_copy(zeros_hbm, acc_sh)

        plsc.subcore_barrier()

        def loop(g, carry):
            base = wid * EPW + g * CH
            pltpu.sync_copy(src_hbm.at[pl.ds(base, CH)], isrc_v)
            pltpu.sync_copy(dst_hbm.at[pl.ds(base, CH)], idst_v)
            pltpu.async_copy(h_hbm.at[isrc_v], rows_v, sem).wait()
            pltpu.sync_copy(rows_v, acc_sh.at[idst_v], add=True)
            return carry

        lax.fori_loop(0, NCH, loop, 0)
        plsc.subcore_barrier()

        @pl.when(s == 0)
        def _drain():
            pltpu.sync_copy(acc_sh, out_hbm.at[c])

    return body


def _sc_degree(dst_p, zeros1):
    return pl.kernel(
        _sc_degree_body,
        out_type=jax.ShapeDtypeStruct((NC * NP,), jnp.float32),
        mesh=_mesh,
        compiler_params=_sc_params,
        scratch_types=[
            pltpu.VMEM((CH,), jnp.int32),
            pltpu.VMEM((CH,), jnp.float32),
            pltpu.VMEM((RPS,), jnp.float32),
            pltpu.VMEM_SHARED((NP,), jnp.float32),
        ],
    )(dst_p, zeros1)


def _sc_scatter(src_p, dst_p, h_p, zeros_d, D):
    return pl.kernel(
        _make_sc_scatter_body(D),
        out_type=jax.ShapeDtypeStruct((NC, NP, D), jnp.float32),
        mesh=_mesh,
        compiler_params=_sc_params,
        scratch_types=[
            pltpu.VMEM((CH,), jnp.int32),
            pltpu.VMEM((CH,), jnp.int32),
            pltpu.VMEM((CH, D), jnp.float32),
            pltpu.VMEM_SHARED((NP, D), jnp.float32),
            pltpu.SemaphoreType.DMA,
        ],
    )(src_p, dst_p, h_p, zeros_d)


def _tc_a_body(xp_ref, w1_ref, degt_ref, h1p_ref, dinv_ref):
    deg = degt_ref[:, 0:1] + degt_ref[:, 1:2] + 1.0
    dinv = lax.rsqrt(deg)
    h1 = jnp.dot(xp_ref[...], w1_ref[...], preferred_element_type=jnp.float32)
    h1p_ref[...] = h1 * dinv
    dinv_ref[...] = dinv


def _tc_b_body(accp_ref, h1p_ref, dinv_ref, b1_ref, w2_ref, h2p_ref):
    dinv = dinv_ref[...]
    agg = dinv * (accp_ref[0] + accp_ref[1] + h1p_ref[...]) + b1_ref[...]
    h = jnp.maximum(agg, 0.0)
    h2 = jnp.dot(h, w2_ref[...], preferred_element_type=jnp.float32)
    mask = (lax.broadcasted_iota(jnp.int32, (NP, 1), 0) < N).astype(jnp.float32)
    h2p_ref[...] = h2 * dinv * mask


def _tc_c_body(accp_ref, h2p_ref, dinv_ref, b2_ref, wfc_ref, bfc_ref, out_ref):
    dinv = dinv_ref[...]
    agg = dinv * (accp_ref[0] + accp_ref[1] + h2p_ref[...]) + b2_ref[...]
    z = jnp.dot(agg, wfc_ref[...], preferred_element_type=jnp.float32) + bfc_ref[...]
    out_ref[...] = jax.nn.sigmoid(z)


def kernel(x, edge_index, W1, b1, W2, b2, Wfc, bfc):
    src = edge_index[0].astype(jnp.int32)
    dst = edge_index[1].astype(jnp.int32)
    padv = N + (jnp.arange(EP - E, dtype=jnp.int32) % (NP - N))
    src_p = jnp.concatenate([src, padv])
    dst_p = jnp.concatenate([dst, padv])

    xp = jnp.pad(x, ((0, NP - N), (0, 0)))
    zeros1 = jnp.zeros((NP,), jnp.float32)
    zeros16 = jnp.zeros((NP, HID), jnp.float32)
    zeros8 = jnp.zeros((NP, OUT), jnp.float32)

    degp = _sc_degree(dst_p, zeros1)
    degt = degp.reshape(NC, NP).T

    h1p, dinv = pl.pallas_call(
        _tc_a_body,
        out_shape=[
            jax.ShapeDtypeStruct((NP, HID), jnp.float32),
            jax.ShapeDtypeStruct((NP, 1), jnp.float32),
        ],
    )(xp, W1, degt)

    accp1 = _sc_scatter(src_p, dst_p, h1p, zeros16, HID)

    h2p = pl.pallas_call(
        _tc_b_body,
        out_shape=jax.ShapeDtypeStruct((NP, OUT), jnp.float32),
    )(accp1, h1p, dinv, b1.reshape(1, HID), W2)

    accp2 = _sc_scatter(src_p, dst_p, h2p, zeros8, OUT)

    outp = pl.pallas_call(
        _tc_c_body,
        out_shape=jax.ShapeDtypeStruct((NP, 1), jnp.float32),
    )(accp2, h2p, dinv, b2.reshape(1, OUT), Wfc, bfc.reshape(1, 1))

    return outp[:N]


# speedup vs baseline: 50.6290x; 2.3306x over previous
import functools

import jax
import jax.numpy as jnp
from jax import lax
from jax.experimental import pallas as pl
from jax.experimental.pallas import tpu as pltpu
from jax.experimental.pallas import tpu_sc as plsc

N = 10000
E = 320000
IN_DIM = 128
HID = 16
OUT = 8

NC = 2
NS = 16
NW = NC * NS
CH = 128
GB = 8
EPW = 10240
EP = NW * EPW
NCH = EPW // CH
NT = NCH // GB
NP = 10240
RPS = NP // NS

_mesh = plsc.VectorSubcoreMesh(
    core_axis_name="c", subcore_axis_name="s", num_cores=NC, num_subcores=NS
)
_sc_params = pltpu.CompilerParams(use_tc_tiling_on_sc=False)


def _sc_degree_body(dst_hbm, zeros_hbm, out_hbm, idx_v, ones_v, slab_v, deg_sh, sem):
    c = lax.axis_index("c")
    s = lax.axis_index("s")
    wid = s * NC + c
    rs = pl.ds(s * RPS, RPS)
    pltpu.sync_copy(zeros_hbm.at[rs], slab_v)
    pltpu.sync_copy(slab_v, deg_sh.at[rs])
    for j in range(CH // 16):
        ones_v[pl.ds(j * 16, 16)] = jnp.ones((16,), jnp.float32)
    plsc.subcore_barrier()

    def loop(t, carry):
        rowbase = wid * NCH + t * GB
        pltpu.sync_copy(dst_hbm.at[pl.ds(rowbase, GB)], idx_v)
        ds = [pltpu.async_copy(ones_v, deg_sh.at[idx_v.at[j]], sem, add=True)
              for j in range(GB)]
        for d in ds:
            d.wait()
        return carry

    lax.fori_loop(0, NT, loop, 0)
    plsc.subcore_barrier()
    pltpu.sync_copy(deg_sh.at[rs], slab_v)
    pltpu.sync_copy(slab_v, out_hbm.at[pl.ds(c * NP + s * RPS, RPS)])


def _make_sc_scatter_body(D):
    def body(src_hbm, dst_hbm, h_hbm, zeros_hbm, out_hbm,
             isrc_v, idst_v, rows_v, acc_sh, gsem, ssem):
        c = lax.axis_index("c")
        s = lax.axis_index("s")
        wid = s * NC + c

        @pl.when(s == 0)
        def _init():
            pltpu.sync_copy(zeros_hbm, acc_sh)

        plsc.subcore_barrier()

        def loop(t, carry):
            rowbase = wid * NCH + t * GB
            pltpu.sync_copy(src_hbm.at[pl.ds(rowbase, GB)], isrc_v)
            pltpu.sync_copy(dst_hbm.at[pl.ds(rowbase, GB)], idst_v)
            gds = [pltpu.async_copy(h_hbm.at[isrc_v.at[j]], rows_v.at[j], gsem)
                   for j in range(GB)]
            for d in gds:
                d.wait()
            sds = [pltpu.async_copy(rows_v.at[j], acc_sh.at[idst_v.at[j]],
                                    ssem, add=True)
                   for j in range(GB)]
            for d in sds:
                d.wait()
            return carry

        lax.fori_loop(0, NT, loop, 0)
        plsc.subcore_barrier()

        @pl.when(s == 0)
        def _drain():
            pltpu.sync_copy(acc_sh, out_hbm.at[c])

    return body


def _sc_degree(dst_p, zeros1):
    return pl.kernel(
        _sc_degree_body,
        out_type=jax.ShapeDtypeStruct((NC * NP,), jnp.float32),
        mesh=_mesh,
        compiler_params=_sc_params,
        scratch_types=[
            pltpu.VMEM((GB, CH), jnp.int32),
            pltpu.VMEM((CH,), jnp.float32),
            pltpu.VMEM((RPS,), jnp.float32),
            pltpu.VMEM_SHARED((NP,), jnp.float32),
            pltpu.SemaphoreType.DMA,
        ],
    )(dst_p, zeros1)


def _sc_scatter(src_p, dst_p, h_p, zeros_d, D):
    return pl.kernel(
        _make_sc_scatter_body(D),
        out_type=jax.ShapeDtypeStruct((NC, NP, D), jnp.float32),
        mesh=_mesh,
        compiler_params=_sc_params,
        scratch_types=[
            pltpu.VMEM((GB, CH), jnp.int32),
            pltpu.VMEM((GB, CH), jnp.int32),
            pltpu.VMEM((GB, CH, D), jnp.float32),
            pltpu.VMEM_SHARED((NP, D), jnp.float32),
            pltpu.SemaphoreType.DMA,
            pltpu.SemaphoreType.DMA,
        ],
    )(src_p, dst_p, h_p, zeros_d)


def _tc_a_body(xp_ref, w1_ref, degt_ref, h1p_ref, dinv_ref):
    deg = degt_ref[:, 0:1] + degt_ref[:, 1:2] + 1.0
    dinv = lax.rsqrt(deg)
    h1 = jnp.dot(xp_ref[...], w1_ref[...], preferred_element_type=jnp.float32)
    h1p_ref[...] = h1 * dinv
    dinv_ref[...] = dinv


def _tc_b_body(accp_ref, h1p_ref, dinv_ref, b1_ref, w2_ref, h2p_ref):
    dinv = dinv_ref[...]
    agg = dinv * (accp_ref[0] + accp_ref[1] + h1p_ref[...]) + b1_ref[...]
    h = jnp.maximum(agg, 0.0)
    h2 = jnp.dot(h, w2_ref[...], preferred_element_type=jnp.float32)
    mask = (lax.broadcasted_iota(jnp.int32, (NP, 1), 0) < N).astype(jnp.float32)
    h2p_ref[...] = h2 * dinv * mask


def _tc_c_body(accp_ref, h2p_ref, dinv_ref, b2_ref, wfc_ref, bfc_ref, out_ref):
    dinv = dinv_ref[...]
    agg = dinv * (accp_ref[0] + accp_ref[1] + h2p_ref[...]) + b2_ref[...]
    z = jnp.dot(agg, wfc_ref[...], preferred_element_type=jnp.float32) + bfc_ref[...]
    out_ref[...] = jax.nn.sigmoid(z)


def kernel(x, edge_index, W1, b1, W2, b2, Wfc, bfc):
    src = edge_index[0].astype(jnp.int32)
    dst = edge_index[1].astype(jnp.int32)
    padv = N + (jnp.arange(EP - E, dtype=jnp.int32) % (NP - N))
    src_p = jnp.concatenate([src, padv]).reshape(EP // CH, CH)
    dst_p = jnp.concatenate([dst, padv]).reshape(EP // CH, CH)

    xp = jnp.pad(x, ((0, NP - N), (0, 0)))
    zeros1 = jnp.zeros((NP,), jnp.float32)
    zeros16 = jnp.zeros((NP, HID), jnp.float32)
    zeros8 = jnp.zeros((NP, OUT), jnp.float32)

    degp = _sc_degree(dst_p, zeros1)
    degt = degp.reshape(NC, NP).T

    h1p, dinv = pl.pallas_call(
        _tc_a_body,
        out_shape=[
            jax.ShapeDtypeStruct((NP, HID), jnp.float32),
            jax.ShapeDtypeStruct((NP, 1), jnp.float32),
        ],
    )(xp, W1, degt)

    accp1 = _sc_scatter(src_p, dst_p, h1p, zeros16, HID)

    h2p = pl.pallas_call(
        _tc_b_body,
        out_shape=jax.ShapeDtypeStruct((NP, OUT), jnp.float32),
    )(accp1, h1p, dinv, b1.reshape(1, HID), W2)

    accp2 = _sc_scatter(src_p, dst_p, h2p, zeros8, OUT)

    outp = pl.pallas_call(
        _tc_c_body,
        out_shape=jax.ShapeDtypeStruct((NP, 1), jnp.float32),
    )(accp2, h2p, dinv, b2.reshape(1, OUT), Wfc, bfc.reshape(1, 1))

    return outp[:N]


# speedup vs baseline: 56.1719x; 1.1095x over previous
import functools

import jax
import jax.numpy as jnp
from jax import lax
from jax.experimental import pallas as pl
from jax.experimental.pallas import tpu as pltpu
from jax.experimental.pallas import tpu_sc as plsc

N = 10000
E = 320000
IN_DIM = 128
HID = 16
OUT = 8

NC = 2
NS = 16
NW = NC * NS
CH = 128
GB = 16
EPW = 10240
EP = NW * EPW
NCH = EPW // CH
NT = NCH // GB
NP = 10240
RPS = NP // NS

_mesh = plsc.VectorSubcoreMesh(
    core_axis_name="c", subcore_axis_name="s", num_cores=NC, num_subcores=NS
)
_sc_params = pltpu.CompilerParams(use_tc_tiling_on_sc=False)


def _sc_degree_body(dst_hbm, zeros_hbm, out_hbm, idx_v, ones_v, slab_v, deg_sh, sem):
    c = lax.axis_index("c")
    s = lax.axis_index("s")
    wid = s * NC + c
    rs = pl.ds(s * RPS, RPS)
    pltpu.sync_copy(zeros_hbm.at[rs], slab_v)
    pltpu.sync_copy(slab_v, deg_sh.at[rs])
    for j in range(CH // 16):
        ones_v[pl.ds(j * 16, 16)] = jnp.ones((16,), jnp.float32)
    plsc.subcore_barrier()

    def loop(t, carry):
        rowbase = wid * NCH + t * GB
        pltpu.sync_copy(dst_hbm.at[pl.ds(rowbase, GB)], idx_v)
        ds = [pltpu.async_copy(ones_v, deg_sh.at[idx_v.at[j]], sem, add=True)
              for j in range(GB)]
        for d in ds:
            d.wait()
        return carry

    lax.fori_loop(0, NT, loop, 0)
    plsc.subcore_barrier()
    pltpu.sync_copy(deg_sh.at[rs], slab_v)
    pltpu.sync_copy(slab_v, out_hbm.at[pl.ds(c * NP + s * RPS, RPS)])


def _make_sc_scatter_body(D):
    def body(src_hbm, dst_hbm, h_hbm, zeros_hbm, out_hbm,
             isrc_v, idst_v, rows_v, acc_sh, gsem, ssem):
        c = lax.axis_index("c")
        s = lax.axis_index("s")
        wid = s * NC + c

        @pl.when(s == 0)
        def _init():
            pltpu.sync_copy(zeros_hbm, acc_sh)

        plsc.subcore_barrier()

        def loop(t, carry):
            rowbase = wid * NCH + t * GB
            pltpu.sync_copy(src_hbm.at[pl.ds(rowbase, GB)], isrc_v)
            pltpu.sync_copy(dst_hbm.at[pl.ds(rowbase, GB)], idst_v)
            gds = [pltpu.async_copy(h_hbm.at[isrc_v.at[j]], rows_v.at[j], gsem)
                   for j in range(GB)]
            for d in gds:
                d.wait()
            sds = [pltpu.async_copy(rows_v.at[j], acc_sh.at[idst_v.at[j]],
                                    ssem, add=True)
                   for j in range(GB)]
            for d in sds:
                d.wait()
            return carry

        lax.fori_loop(0, NT, loop, 0)
        plsc.subcore_barrier()

        @pl.when(s == 0)
        def _drain():
            pltpu.sync_copy(acc_sh, out_hbm.at[c])

    return body


def _sc_degree(dst_p, zeros1):
    return pl.kernel(
        _sc_degree_body,
        out_type=jax.ShapeDtypeStruct((NC * NP,), jnp.float32),
        mesh=_mesh,
        compiler_params=_sc_params,
        scratch_types=[
            pltpu.VMEM((GB, CH), jnp.int32),
            pltpu.VMEM((CH,), jnp.float32),
            pltpu.VMEM((RPS,), jnp.float32),
            pltpu.VMEM_SHARED((NP,), jnp.float32),
            pltpu.SemaphoreType.DMA,
        ],
    )(dst_p, zeros1)


def _sc_scatter(src_p, dst_p, h_p, zeros_d, D):
    return pl.kernel(
        _make_sc_scatter_body(D),
        out_type=jax.ShapeDtypeStruct((NC, NP, D), jnp.float32),
        mesh=_mesh,
        compiler_params=_sc_params,
        scratch_types=[
            pltpu.VMEM((GB, CH), jnp.int32),
            pltpu.VMEM((GB, CH), jnp.int32),
            pltpu.VMEM((GB, CH, D), jnp.float32),
            pltpu.VMEM_SHARED((NP, D), jnp.float32),
            pltpu.SemaphoreType.DMA,
            pltpu.SemaphoreType.DMA,
        ],
    )(src_p, dst_p, h_p, zeros_d)


def _tc_a_body(xp_ref, w1_ref, degt_ref, h1p_ref, dinv_ref):
    deg = degt_ref[:, 0:1] + degt_ref[:, 1:2] + 1.0
    dinv = lax.rsqrt(deg)
    h1 = jnp.dot(xp_ref[...], w1_ref[...], preferred_element_type=jnp.float32)
    h1p_ref[...] = h1 * dinv
    dinv_ref[...] = dinv


def _tc_b_body(accp_ref, h1p_ref, dinv_ref, b1_ref, w2_ref, h2p_ref):
    dinv = dinv_ref[...]
    agg = dinv * (accp_ref[0] + accp_ref[1] + h1p_ref[...]) + b1_ref[...]
    h = jnp.maximum(agg, 0.0)
    h2 = jnp.dot(h, w2_ref[...], preferred_element_type=jnp.float32)
    mask = (lax.broadcasted_iota(jnp.int32, (NP, 1), 0) < N).astype(jnp.float32)
    h2p_ref[...] = h2 * dinv * mask


def _tc_c_body(accp_ref, h2p_ref, dinv_ref, b2_ref, wfc_ref, bfc_ref, out_ref):
    dinv = dinv_ref[...]
    agg = dinv * (accp_ref[0] + accp_ref[1] + h2p_ref[...]) + b2_ref[...]
    z = jnp.dot(agg, wfc_ref[...], preferred_element_type=jnp.float32) + bfc_ref[...]
    out_ref[...] = jax.nn.sigmoid(z)


def kernel(x, edge_index, W1, b1, W2, b2, Wfc, bfc):
    src = edge_index[0].astype(jnp.int32)
    dst = edge_index[1].astype(jnp.int32)
    padv = N + (jnp.arange(EP - E, dtype=jnp.int32) % (NP - N))
    src_p = jnp.concatenate([src, padv]).reshape(EP // CH, CH)
    dst_p = jnp.concatenate([dst, padv]).reshape(EP // CH, CH)

    xp = jnp.pad(x, ((0, NP - N), (0, 0)))
    zeros1 = jnp.zeros((NP,), jnp.float32)
    zeros16 = jnp.zeros((NP, HID), jnp.float32)
    zeros8 = jnp.zeros((NP, OUT), jnp.float32)

    degp = _sc_degree(dst_p, zeros1)
    degt = degp.reshape(NC, NP).T

    h1p, dinv = pl.pallas_call(
        _tc_a_body,
        out_shape=[
            jax.ShapeDtypeStruct((NP, HID), jnp.float32),
            jax.ShapeDtypeStruct((NP, 1), jnp.float32),
        ],
    )(xp, W1, degt)

    accp1 = _sc_scatter(src_p, dst_p, h1p, zeros16, HID)

    h2p = pl.pallas_call(
        _tc_b_body,
        out_shape=jax.ShapeDtypeStruct((NP, OUT), jnp.float32),
    )(accp1, h1p, dinv, b1.reshape(1, HID), W2)

    accp2 = _sc_scatter(src_p, dst_p, h2p, zeros8, OUT)

    outp = pl.pallas_call(
        _tc_c_body,
        out_shape=jax.ShapeDtypeStruct((NP, 1), jnp.float32),
    )(accp2, h2p, dinv, b2.reshape(1, OUT), Wfc, bfc.reshape(1, 1))

    return outp[:N]


# speedup vs baseline: 61.8938x; 1.1019x over previous
import functools

import jax
import jax.numpy as jnp
from jax import lax
from jax.experimental import pallas as pl
from jax.experimental.pallas import tpu as pltpu
from jax.experimental.pallas import tpu_sc as plsc

N = 10000
E = 320000
IN_DIM = 128
HID = 16
OUT = 8

NC = 2
NS = 16
NW = NC * NS
CH = 128
GB = 16
EPW = 10240
EP = NW * EPW
NCH = EPW // CH
NT = NCH // GB
NP = 10240
RPS = NP // NS

_mesh = plsc.VectorSubcoreMesh(
    core_axis_name="c", subcore_axis_name="s", num_cores=NC, num_subcores=NS
)
_sc_params = pltpu.CompilerParams(use_tc_tiling_on_sc=False)


def _sc_degree_body(dst_hbm, zeros_hbm, out_hbm, idx_v, ones_v, slab_v, deg_sh, sem):
    c = lax.axis_index("c")
    s = lax.axis_index("s")
    wid = s * NC + c
    rs = pl.ds(s * RPS, RPS)
    pltpu.sync_copy(zeros_hbm.at[rs], slab_v)
    pltpu.sync_copy(slab_v, deg_sh.at[rs])
    for j in range(CH // 16):
        ones_v[pl.ds(j * 16, 16)] = jnp.ones((16,), jnp.float32)
    plsc.subcore_barrier()

    def loop(t, carry):
        rowbase = wid * NCH + t * GB
        pltpu.sync_copy(dst_hbm.at[pl.ds(rowbase, GB)], idx_v)
        ds = [pltpu.async_copy(ones_v, deg_sh.at[idx_v.at[j]], sem, add=True)
              for j in range(GB)]
        for d in ds:
            d.wait()
        return carry

    lax.fori_loop(0, NT, loop, 0)
    plsc.subcore_barrier()
    pltpu.sync_copy(deg_sh.at[rs], slab_v)
    pltpu.sync_copy(slab_v, out_hbm.at[pl.ds(c * NP + s * RPS, RPS)])


def _make_sc_scatter_body(D):
    def body(src_hbm, dst_hbm, h_hbm, zeros_hbm, out_hbm,
             isrc_v, idst_v, rows_v, h_sh, acc_sh, gsem, ssem):
        c = lax.axis_index("c")
        s = lax.axis_index("s")
        wid = s * NC + c
        rs = pl.ds(s * RPS, RPS)

        @pl.when(s == 0)
        def _init():
            pltpu.sync_copy(zeros_hbm, acc_sh)

        pltpu.sync_copy(h_hbm.at[rs], h_sh.at[rs])
        plsc.subcore_barrier()

        def loop(t, carry):
            rowbase = wid * NCH + t * GB
            pltpu.sync_copy(src_hbm.at[pl.ds(rowbase, GB)], isrc_v)
            pltpu.sync_copy(dst_hbm.at[pl.ds(rowbase, GB)], idst_v)
            gds = [pltpu.async_copy(h_sh.at[isrc_v.at[j]], rows_v.at[j], gsem)
                   for j in range(GB)]
            for d in gds:
                d.wait()
            sds = [pltpu.async_copy(rows_v.at[j], acc_sh.at[idst_v.at[j]],
                                    ssem, add=True)
                   for j in range(GB)]
            for d in sds:
                d.wait()
            return carry

        lax.fori_loop(0, NT, loop, 0)
        plsc.subcore_barrier()

        @pl.when(s == 0)
        def _drain():
            pltpu.sync_copy(acc_sh, out_hbm.at[c])

    return body


def _sc_degree(dst_p, zeros1):
    return pl.kernel(
        _sc_degree_body,
        out_type=jax.ShapeDtypeStruct((NC * NP,), jnp.float32),
        mesh=_mesh,
        compiler_params=_sc_params,
        scratch_types=[
            pltpu.VMEM((GB, CH), jnp.int32),
            pltpu.VMEM((CH,), jnp.float32),
            pltpu.VMEM((RPS,), jnp.float32),
            pltpu.VMEM_SHARED((NP,), jnp.float32),
            pltpu.SemaphoreType.DMA,
        ],
    )(dst_p, zeros1)


def _sc_scatter(src_p, dst_p, h_p, zeros_d, D):
    return pl.kernel(
        _make_sc_scatter_body(D),
        out_type=jax.ShapeDtypeStruct((NC, NP, D), jnp.float32),
        mesh=_mesh,
        compiler_params=_sc_params,
        scratch_types=[
            pltpu.VMEM((GB, CH), jnp.int32),
            pltpu.VMEM((GB, CH), jnp.int32),
            pltpu.VMEM((GB, CH, D), jnp.float32),
            pltpu.VMEM_SHARED((NP, D), jnp.float32),
            pltpu.VMEM_SHARED((NP, D), jnp.float32),
            pltpu.SemaphoreType.DMA,
            pltpu.SemaphoreType.DMA,
        ],
    )(src_p, dst_p, h_p, zeros_d)


def _tc_a_body(xp_ref, w1_ref, degt_ref, h1p_ref, dinv_ref):
    deg = degt_ref[:, 0:1] + degt_ref[:, 1:2] + 1.0
    dinv = lax.rsqrt(deg)
    h1 = jnp.dot(xp_ref[...], w1_ref[...], preferred_element_type=jnp.float32)
    h1p_ref[...] = h1 * dinv
    dinv_ref[...] = dinv


def _tc_b_body(accp_ref, h1p_ref, dinv_ref, b1_ref, w2_ref, h2p_ref):
    dinv = dinv_ref[...]
    agg = dinv * (accp_ref[0] + accp_ref[1] + h1p_ref[...]) + b1_ref[...]
    h = jnp.maximum(agg, 0.0)
    h2 = jnp.dot(h, w2_ref[...], preferred_element_type=jnp.float32)
    mask = (lax.broadcasted_iota(jnp.int32, (NP, 1), 0) < N).astype(jnp.float32)
    h2p_ref[...] = h2 * dinv * mask


def _tc_c_body(accp_ref, h2p_ref, dinv_ref, b2_ref, wfc_ref, bfc_ref, out_ref):
    dinv = dinv_ref[...]
    agg = dinv * (accp_ref[0] + accp_ref[1] + h2p_ref[...]) + b2_ref[...]
    z = jnp.dot(agg, wfc_ref[...], preferred_element_type=jnp.float32) + bfc_ref[...]
    out_ref[...] = jax.nn.sigmoid(z)


def kernel(x, edge_index, W1, b1, W2, b2, Wfc, bfc):
    src = edge_index[0].astype(jnp.int32)
    dst = edge_index[1].astype(jnp.int32)
    padv = N + (jnp.arange(EP - E, dtype=jnp.int32) % (NP - N))
    src_p = jnp.concatenate([src, padv]).reshape(EP // CH, CH)
    dst_p = jnp.concatenate([dst, padv]).reshape(EP // CH, CH)

    xp = jnp.pad(x, ((0, NP - N), (0, 0)))
    zeros1 = jnp.zeros((NP,), jnp.float32)
    zeros16 = jnp.zeros((NP, HID), jnp.float32)
    zeros8 = jnp.zeros((NP, OUT), jnp.float32)

    degp = _sc_degree(dst_p, zeros1)
    degt = degp.reshape(NC, NP).T

    h1p, dinv = pl.pallas_call(
        _tc_a_body,
        out_shape=[
            jax.ShapeDtypeStruct((NP, HID), jnp.float32),
            jax.ShapeDtypeStruct((NP, 1), jnp.float32),
        ],
    )(xp, W1, degt)

    accp1 = _sc_scatter(src_p, dst_p, h1p, zeros16, HID)

    h2p = pl.pallas_call(
        _tc_b_body,
        out_shape=jax.ShapeDtypeStruct((NP, OUT), jnp.float32),
    )(accp1, h1p, dinv, b1.reshape(1, HID), W2)

    accp2 = _sc_scatter(src_p, dst_p, h2p, zeros8, OUT)

    outp = pl.pallas_call(
        _tc_c_body,
        out_shape=jax.ShapeDtypeStruct((NP, 1), jnp.float32),
    )(accp2, h2p, dinv, b2.reshape(1, OUT), Wfc, bfc.reshape(1, 1))

    return outp[:N]


# speedup vs baseline: 61.9121x; 1.0003x over previous
import functools

import jax
import jax.numpy as jnp
from jax import lax
from jax.experimental import pallas as pl
from jax.experimental.pallas import tpu as pltpu
from jax.experimental.pallas import tpu_sc as plsc

N = 10000
E = 320000
IN_DIM = 128
HID = 16
OUT = 8

NC = 2
NS = 16
NW = NC * NS
CH = 128
GB = 16
EPW = 10240
EP = NW * EPW
NCH = EPW // CH
NT = NCH // GB
NP = 10240
RPS = NP // NS

_mesh = plsc.VectorSubcoreMesh(
    core_axis_name="c", subcore_axis_name="s", num_cores=NC, num_subcores=NS
)
_sc_params = pltpu.CompilerParams(
    use_tc_tiling_on_sc=False, skip_device_barrier=True
)


def _sc_degree_body(dst_hbm, zeros_hbm, out_hbm, idx_v, ones_v, slab_v, deg_sh, sem):
    c = lax.axis_index("c")
    s = lax.axis_index("s")
    wid = s * NC + c
    rs = pl.ds(s * RPS, RPS)
    pltpu.sync_copy(zeros_hbm.at[rs], slab_v)
    pltpu.sync_copy(slab_v, deg_sh.at[rs])
    for j in range(CH // 16):
        ones_v[pl.ds(j * 16, 16)] = jnp.ones((16,), jnp.float32)
    plsc.subcore_barrier()

    def loop(t, carry):
        rowbase = wid * NCH + t * GB
        pltpu.sync_copy(dst_hbm.at[pl.ds(rowbase, GB)], idx_v)
        ds = [pltpu.async_copy(ones_v, deg_sh.at[idx_v.at[j]], sem, add=True)
              for j in range(GB)]
        for d in ds:
            d.wait()
        return carry

    lax.fori_loop(0, NT, loop, 0)
    plsc.subcore_barrier()
    pltpu.sync_copy(deg_sh.at[rs], slab_v)
    pltpu.sync_copy(slab_v, out_hbm.at[pl.ds(c * NP + s * RPS, RPS)])


def _make_sc_scatter_body(D):
    def body(src_hbm, dst_hbm, h_hbm, zeros_hbm, out_hbm,
             isrc_v, idst_v, rows_v, h_sh, acc_sh, gsem, ssem):
        c = lax.axis_index("c")
        s = lax.axis_index("s")
        wid = s * NC + c
        rs = pl.ds(s * RPS, RPS)

        @pl.when(s == 0)
        def _init():
            pltpu.sync_copy(zeros_hbm, acc_sh)

        pltpu.sync_copy(h_hbm.at[rs], h_sh.at[rs])
        plsc.subcore_barrier()

        def loop(t, carry):
            rowbase = wid * NCH + t * GB
            pltpu.sync_copy(src_hbm.at[pl.ds(rowbase, GB)], isrc_v)
            pltpu.sync_copy(dst_hbm.at[pl.ds(rowbase, GB)], idst_v)
            gds = [pltpu.async_copy(h_sh.at[isrc_v.at[j]], rows_v.at[j], gsem)
                   for j in range(GB)]
            for d in gds:
                d.wait()
            sds = [pltpu.async_copy(rows_v.at[j], acc_sh.at[idst_v.at[j]],
                                    ssem, add=True)
                   for j in range(GB)]
            for d in sds:
                d.wait()
            return carry

        lax.fori_loop(0, NT, loop, 0)
        plsc.subcore_barrier()

        @pl.when(s == 0)
        def _drain():
            pltpu.sync_copy(acc_sh, out_hbm.at[c])

    return body


def _sc_degree(dst_p, zeros1):
    return pl.kernel(
        _sc_degree_body,
        out_type=jax.ShapeDtypeStruct((NC * NP,), jnp.float32),
        mesh=_mesh,
        compiler_params=_sc_params,
        scratch_types=[
            pltpu.VMEM((GB, CH), jnp.int32),
            pltpu.VMEM((CH,), jnp.float32),
            pltpu.VMEM((RPS,), jnp.float32),
            pltpu.VMEM_SHARED((NP,), jnp.float32),
            pltpu.SemaphoreType.DMA,
        ],
    )(dst_p, zeros1)


def _sc_scatter(src_p, dst_p, h_p, zeros_d, D):
    return pl.kernel(
        _make_sc_scatter_body(D),
        out_type=jax.ShapeDtypeStruct((NC, NP, D), jnp.float32),
        mesh=_mesh,
        compiler_params=_sc_params,
        scratch_types=[
            pltpu.VMEM((GB, CH), jnp.int32),
            pltpu.VMEM((GB, CH), jnp.int32),
            pltpu.VMEM((GB, CH, D), jnp.float32),
            pltpu.VMEM_SHARED((NP, D), jnp.float32),
            pltpu.VMEM_SHARED((NP, D), jnp.float32),
            pltpu.SemaphoreType.DMA,
            pltpu.SemaphoreType.DMA,
        ],
    )(src_p, dst_p, h_p, zeros_d)


def _tc_a_body(xp_ref, w1_ref, degt_ref, h1p_ref, dinv_ref):
    deg = degt_ref[:, 0:1] + degt_ref[:, 1:2] + 1.0
    dinv = lax.rsqrt(deg)
    h1 = jnp.dot(xp_ref[...], w1_ref[...], preferred_element_type=jnp.float32)
    h1p_ref[...] = h1 * dinv
    dinv_ref[...] = dinv


def _tc_b_body(accp_ref, h1p_ref, dinv_ref, b1_ref, w2_ref, h2p_ref):
    dinv = dinv_ref[...]
    agg = dinv * (accp_ref[0] + accp_ref[1] + h1p_ref[...]) + b1_ref[...]
    h = jnp.maximum(agg, 0.0)
    h2 = jnp.dot(h, w2_ref[...], preferred_element_type=jnp.float32)
    mask = (lax.broadcasted_iota(jnp.int32, (NP, 1), 0) < N).astype(jnp.float32)
    h2p_ref[...] = h2 * dinv * mask


def _tc_c_body(accp_ref, h2p_ref, dinv_ref, b2_ref, wfc_ref, bfc_ref, out_ref):
    dinv = dinv_ref[...]
    agg = dinv * (accp_ref[0] + accp_ref[1] + h2p_ref[...]) + b2_ref[...]
    z = jnp.dot(agg, wfc_ref[...], preferred_element_type=jnp.float32) + bfc_ref[...]
    out_ref[...] = jax.nn.sigmoid(z)


def kernel(x, edge_index, W1, b1, W2, b2, Wfc, bfc):
    src = edge_index[0].astype(jnp.int32)
    dst = edge_index[1].astype(jnp.int32)
    padv = N + (jnp.arange(EP - E, dtype=jnp.int32) % (NP - N))
    src_p = jnp.concatenate([src, padv]).reshape(EP // CH, CH)
    dst_p = jnp.concatenate([dst, padv]).reshape(EP // CH, CH)

    xp = jnp.pad(x, ((0, NP - N), (0, 0)))
    zeros1 = jnp.zeros((NP,), jnp.float32)
    zeros16 = jnp.zeros((NP, HID), jnp.float32)
    zeros8 = jnp.zeros((NP, OUT), jnp.float32)

    degp = _sc_degree(dst_p, zeros1)
    degt = degp.reshape(NC, NP).T

    h1p, dinv = pl.pallas_call(
        _tc_a_body,
        out_shape=[
            jax.ShapeDtypeStruct((NP, HID), jnp.float32),
            jax.ShapeDtypeStruct((NP, 1), jnp.float32),
        ],
    )(xp, W1, degt)

    accp1 = _sc_scatter(src_p, dst_p, h1p, zeros16, HID)

    h2p = pl.pallas_call(
        _tc_b_body,
        out_shape=jax.ShapeDtypeStruct((NP, OUT), jnp.float32),
    )(accp1, h1p, dinv, b1.reshape(1, HID), W2)

    accp2 = _sc_scatter(src_p, dst_p, h2p, zeros8, OUT)

    outp = pl.pallas_call(
        _tc_c_body,
        out_shape=jax.ShapeDtypeStruct((NP, 1), jnp.float32),
    )(accp2, h2p, dinv, b2.reshape(1, OUT), Wfc, bfc.reshape(1, 1))

    return outp[:N]
